# Initial kernel scaffold; baseline (speedup 1.0000x reference)
import jax
import jax.numpy as jnp
from jax.experimental import pallas as pl


def kernel(x, modality, w_gates, W1, b1, W2, b2, Wout, bout):
    raise NotImplementedError("write your pallas kernel here")


# speedup vs baseline: 2.6773x; 2.6773x over previous
import jax
import jax.numpy as jnp
from jax.experimental import pallas as pl

IN_DIM = 1024
OUT_DIM = 1000
NUM_EXPERT = 16
TOP_K = 12
HIDDEN = 256
N_TOK = 2048
TILE_N = 256


def _moe_kernel(x_ref, wg_ref, W1_ref, W2_ref, Wout_ref, bout_ref,
                y_ref, gates_ref, load_ref):
    i = pl.program_id(0)
    x = x_ref[...]

    logits = jnp.dot(x, wg_ref[...], preferred_element_type=jnp.float32)
    iota_j = jax.lax.broadcasted_iota(jnp.int32, (TILE_N, NUM_EXPERT), 1)
    sel = jnp.zeros((TILE_N, NUM_EXPERT), jnp.float32)
    for e in range(NUM_EXPERT):
        col = logits[:, e:e + 1]
        rank = jnp.sum(
            (logits > col).astype(jnp.int32)
            + ((logits == col) & (iota_j < e)).astype(jnp.int32),
            axis=1, keepdims=True)
        onehot = (iota_j == e).astype(jnp.float32)
        sel = sel + jnp.where(rank < TOP_K, 1.0, 0.0) * onehot
    m = jnp.max(logits, axis=1, keepdims=True)
    ex = jnp.where(sel > 0.0, jnp.exp(logits - m), 0.0)
    g = ex / jnp.sum(ex, axis=1, keepdims=True)
    gates_ref[...] = g

    @pl.when(i == 0)
    def _():
        load_ref[...] = jnp.zeros_like(load_ref)
    load_ref[...] += jnp.sum((g > 0).astype(jnp.float32), axis=0,
                             keepdims=True)

    acc = jnp.zeros((TILE_N, IN_DIM), jnp.float32)
    for e in range(NUM_EXPERT):
        h = jnp.maximum(
            jnp.dot(x, W1_ref[e], preferred_element_type=jnp.float32), 0.0)
        hg = h * g[:, e:e + 1]
        acc = acc + jnp.dot(hg, W2_ref[e], preferred_element_type=jnp.float32)

    yin = jnp.maximum(acc, 0.0) + x
    y_ref[...] = (jnp.dot(yin, Wout_ref[...], preferred_element_type=jnp.float32)
                  + bout_ref[...])


def kernel(x, modality, w_gates, W1, b1, W2, b2, Wout, bout):
    wg = w_gates[modality]
    n_tiles = N_TOK // TILE_N
    y, gates, load = pl.pallas_call(
        _moe_kernel,
        grid=(n_tiles,),
        in_specs=[
            pl.BlockSpec((TILE_N, IN_DIM), lambda i: (i, 0)),
            pl.BlockSpec((IN_DIM, NUM_EXPERT), lambda i: (0, 0)),
            pl.BlockSpec((NUM_EXPERT, IN_DIM, HIDDEN), lambda i: (0, 0, 0)),
            pl.BlockSpec((NUM_EXPERT, HIDDEN, IN_DIM), lambda i: (0, 0, 0)),
            pl.BlockSpec((IN_DIM, OUT_DIM), lambda i: (0, 0)),
            pl.BlockSpec((1, OUT_DIM), lambda i: (0, 0)),
        ],
        out_specs=[
            pl.BlockSpec((TILE_N, OUT_DIM), lambda i: (i, 0)),
            pl.BlockSpec((TILE_N, NUM_EXPERT), lambda i: (i, 0)),
            pl.BlockSpec((1, NUM_EXPERT), lambda i: (0, 0)),
        ],
        out_shape=[
            jax.ShapeDtypeStruct((N_TOK, OUT_DIM), jnp.float32),
            jax.ShapeDtypeStruct((N_TOK, NUM_EXPERT), jnp.float32),
            jax.ShapeDtypeStruct((1, NUM_EXPERT), jnp.float32),
        ],
    )(x, wg, W1, W2, Wout, bout.reshape(1, OUT_DIM))
    return (y, gates, load.reshape(NUM_EXPERT))
